# baseline (speedup 1.0000x reference)
import functools

import jax
import jax.numpy as jnp
from jax import lax
from jax.experimental import pallas as pl
from jax.experimental.pallas import tpu as pltpu
from jax.experimental.pallas import tpu_sc as plsc

N = 10000
E = 320000
D = 128
C = 40
G = 64

NC = 2
NS = 16
NW = NC * NS
EDGES_PER_TILE = E // NW
CHUNK = 80
NCHUNKS = EDGES_PER_TILE // CHUNK
ROW0 = 640
STAGE = 80
NSTAGE0 = ROW0 // STAGE

_MESH = plsc.VectorSubcoreMesh(core_axis_name="c", subcore_axis_name="s")


def _zero_fill(ref, rows, width):
    def body(i, _):
        for j in range(width // 16):
            ref[i, pl.ds(j * 16, 16)] = jnp.zeros((16,), jnp.float32)
        return 0
    lax.fori_loop(0, rows, body, 0, unroll=False)


NPAD = NS * ROW0


def _deg_body(dst_hbm, out_lo, out_hi, idxv, hist, stage_v, sh):
    c = lax.axis_index("c")
    s = lax.axis_index("s")
    w = c * NS + s

    def z(i, _):
        hist[pl.ds(i * 16, 16)] = jnp.zeros((16,), jnp.float32)
        return 0
    lax.fori_loop(0, NPAD // 16, z, 0, unroll=False)

    pltpu.sync_copy(dst_hbm.at[pl.ds(w * EDGES_PER_TILE, EDGES_PER_TILE)], idxv)
    ones16 = jnp.ones((16,), jnp.float32)
    def body(i, _):
        v = idxv[pl.ds(i * 16, 16)]
        plsc.addupdate_scatter(hist, [v], ones16)
        return 0
    lax.fori_loop(0, EDGES_PER_TILE // 16, body, 0, unroll=False)

    pltpu.sync_copy(hist, sh.at[pl.ds(s * NPAD, NPAD)])
    plsc.subcore_barrier()
    for t in range(NS):
        pltpu.sync_copy(sh.at[pl.ds(t * NPAD + s * ROW0, ROW0)],
                        hist.at[pl.ds(t * ROW0, ROW0)])
    def red(j, _):
        acc = hist[pl.ds(j * 16, 16)]
        for t in range(1, NS):
            acc = acc + hist[pl.ds(t * ROW0 + j * 16, 16)]
        stage_v[pl.ds(j * 16, 16)] = acc
        return 0
    lax.fori_loop(0, ROW0 // 16, red, 0, unroll=False)

    @pl.when(c == 0)
    def _():
        pltpu.sync_copy(stage_v, out_lo.at[pl.ds(s * ROW0, ROW0)])

    @pl.when(c == 1)
    def _():
        pltpu.sync_copy(stage_v, out_hi.at[pl.ds(s * ROW0, ROW0)])


_deg_kernel = functools.partial(
    pl.kernel,
    mesh=_MESH,
    compiler_params=pltpu.CompilerParams(needs_layout_passes=False),
    out_type=[jax.ShapeDtypeStruct((NPAD,), jnp.float32),
              jax.ShapeDtypeStruct((NPAD,), jnp.float32)],
    scratch_types=[
        pltpu.VMEM((EDGES_PER_TILE,), jnp.int32),
        pltpu.VMEM((NPAD,), jnp.float32),
        pltpu.VMEM((ROW0,), jnp.float32),
        pltpu.VMEM_SHARED((NS * NPAD,), jnp.float32),
    ],
)(_deg_body)


def _agg_body(y_hbm, src3_hbm, dst3_hbm, out_hbm,
              srcv0, srcv1, srcv2, srcv3, dstv0, dstv1, dstv2, dstv3,
              rows0, rows1, rows2, rows3, agg_sh,
              gsem0, gsem1, gsem2, gsem3, ssem0, ssem1, ssem2, ssem3,
              vsem0, vsem1, vsem2, vsem3, dsem0, dsem1, dsem2, dsem3):
    c = lax.axis_index("c")
    s = lax.axis_index("s")
    w = c * NS + s
    nstage = jnp.where(s == NS - 1, 5, NSTAGE0)

    _zero_fill(rows0, STAGE, D)
    def zc(k, _):
        pltpu.sync_copy(rows0, agg_sh.at[pl.ds(s * ROW0 + k * STAGE, STAGE)])
        return 0
    lax.fori_loop(0, nstage, zc, 0, unroll=False)
    plsc.subcore_barrier()

    rows = (rows0, rows1, rows2, rows3)
    srcv = (srcv0, srcv1, srcv2, srcv3)
    dstv = (dstv0, dstv1, dstv2, dstv3)
    gs = (gsem0, gsem1, gsem2, gsem3)
    ss = (ssem0, ssem1, ssem2, ssem3)
    vs = (vsem0, vsem1, vsem2, vsem3)
    dsm = (dsem0, dsem1, dsem2, dsem3)

    LAST = NCHUNKS - 1
    def sv_start(k, b):
        pltpu.async_copy(src3_hbm.at[w, k, 0], srcv[b], vs[b])
    def sv_wait(b):
        pltpu.make_async_copy(src3_hbm.at[w, 0, 0], srcv[b], vs[b]).wait()
    def dv_start(k, b):
        pltpu.async_copy(dst3_hbm.at[w, k, 0], dstv[b], dsm[b])
    def dv_wait(b):
        pltpu.make_async_copy(dst3_hbm.at[w, 0, 0], dstv[b], dsm[b]).wait()
    def g_start(b):
        pltpu.async_copy(y_hbm.at[srcv[b]], rows[b], gs[b])
    def g_wait(b):
        pltpu.make_async_copy(y_hbm.at[srcv[b]], rows[b], gs[b]).wait()
    def s_start(b):
        pltpu.async_copy(rows[b], agg_sh.at[dstv[b]], ss[b], add=True)
    def s_wait(b):
        pltpu.make_async_copy(rows[b], agg_sh.at[dstv[b]], ss[b]).wait()

    sv_start(0, 0); sv_start(1, 1); sv_start(2, 2); sv_start(3, 3)
    dv_start(0, 0); dv_start(1, 1); dv_start(2, 2)
    sv_wait(0); g_start(0)
    sv_wait(1); g_start(1)
    sv_wait(2); g_start(2)
    g_wait(0); sv_start(4, 0); dv_wait(0); s_start(0)
    dv_start(3, 3); sv_wait(3); g_start(3)
    g_wait(1); sv_start(5, 1); dv_wait(1); s_start(1)
    s_wait(0); dv_start(4, 0); sv_wait(0); g_start(0)
    g_wait(2); sv_start(6, 2); dv_wait(2); s_start(2)
    s_wait(1); dv_start(5, 1); sv_wait(1); g_start(1)
    g_wait(3); sv_start(7, 3); dv_wait(3); s_start(3)
    s_wait(2); dv_start(6, 2); sv_wait(2); g_start(2)

    def quad(p, _):
        for b in (0, 1, 2, 3):
            k = 4 * p + b
            bn = (b + 3) % 4
            g_wait(b); sv_start(k + 4, b); dv_wait(b); s_start(b)
            s_wait(bn); dv_start(k + 3, bn); sv_wait(bn); g_start(bn)
        return 0
    lax.fori_loop(1, 30, quad, 0, unroll=False)

    g_wait(0); sv_start(124, 0); dv_wait(0); s_start(0)
    s_wait(3); dv_start(123, 3); sv_wait(3); g_start(3)
    g_wait(1); dv_wait(1); s_start(1)
    s_wait(0); dv_start(124, 0); sv_wait(0); g_start(0)
    g_wait(2); dv_wait(2); s_start(2); s_wait(1)
    g_wait(3); dv_wait(3); s_start(3); s_wait(2)
    g_wait(0); dv_wait(0); s_start(0); s_wait(3)
    s_wait(0)
    plsc.subcore_barrier()

    def out_body(k, _):
        r0 = s * ROW0 + k * STAGE
        pltpu.sync_copy(agg_sh.at[pl.ds(r0, STAGE)], rows0)
        pltpu.sync_copy(rows0, out_hbm.at[pl.ds(c * N + r0, STAGE)])
        return 0
    lax.fori_loop(0, nstage, out_body, 0, unroll=False)


_agg_kernel = functools.partial(
    pl.kernel,
    mesh=_MESH,
    out_type=jax.ShapeDtypeStruct((NC * N, D), jnp.float32),
    scratch_types=(
        [pltpu.VMEM((CHUNK,), jnp.int32) for _ in range(8)]
        + [pltpu.VMEM((CHUNK, D), jnp.float32) for _ in range(4)]
        + [pltpu.VMEM_SHARED((N, D), jnp.float32)]
        + [pltpu.SemaphoreType.DMA for _ in range(16)]
    ),
)(_agg_body)


RB = 1000
NRB = N // RB

_f32 = jnp.float32
_HI = lax.Precision.HIGHEST


def _prep_body(deg_lo, deg_hi, x_ref, w_ref, y_out, dinv_out):
    deg = deg_lo[...] + deg_hi[...] + 1.0
    dinv = lax.rsqrt(deg)
    xw = lax.dot_general(x_ref[...], w_ref[...], (((1,), (1,)), ((), ())),
                         preferred_element_type=_f32, precision=_HI)
    y_out[...] = xw * dinv
    dinv_out[...] = dinv


def _prep_tc(deg_lo, deg_hi, x, W1):
    return pl.pallas_call(
        _prep_body,
        grid=(NRB,),
        in_specs=[
            pl.BlockSpec((RB, 1), lambda i: (i, 0)),
            pl.BlockSpec((RB, 1), lambda i: (i, 0)),
            pl.BlockSpec((RB, D), lambda i: (i, 0)),
            pl.BlockSpec((D, D), lambda i: (0, 0)),
        ],
        out_specs=[
            pl.BlockSpec((RB, D), lambda i: (i, 0)),
            pl.BlockSpec((RB, 1), lambda i: (i, 0)),
        ],
        out_shape=[
            jax.ShapeDtypeStruct((N, D), _f32),
            jax.ShapeDtypeStruct((N, 1), _f32),
        ],
    )(deg_lo, deg_hi, x, W1)


def _mid_body(y_ref, agg_lo, agg_hi, dinv_ref, b_ref, w_ref, out_ref):
    dinv = dinv_ref[...]
    h = dinv * (agg_lo[...] + agg_hi[...] + y_ref[...]) + b_ref[...]
    h = jnp.maximum(h, 0.0)
    xw = lax.dot_general(h, w_ref[...], (((1,), (1,)), ((), ())),
                         preferred_element_type=_f32, precision=_HI)
    out_ref[...] = xw * dinv


def _mid_tc(y, agg, dinv, b, Wn):
    return pl.pallas_call(
        _mid_body,
        grid=(NRB,),
        in_specs=[
            pl.BlockSpec((RB, D), lambda i: (i, 0)),
            pl.BlockSpec((RB, D), lambda i: (i, 0)),
            pl.BlockSpec((RB, D), lambda i: (i + NRB, 0)),
            pl.BlockSpec((RB, 1), lambda i: (i, 0)),
            pl.BlockSpec((1, D), lambda i: (0, 0)),
            pl.BlockSpec((D, D), lambda i: (0, 0)),
        ],
        out_specs=pl.BlockSpec((RB, D), lambda i: (i, 0)),
        out_shape=jax.ShapeDtypeStruct((N, D), _f32),
    )(y, agg, agg, dinv, b, Wn)


def _final_body(y_ref, agg_lo, agg_hi, dinv_ref, b_ref, bv_ref, wl_ref,
                bl_ref, out_ref, sums, cnts):
    i = pl.program_id(0)

    @pl.when(i == 0)
    def _():
        sums[...] = jnp.zeros_like(sums)
        cnts[...] = jnp.zeros_like(cnts)

    h = dinv_ref[...] * (agg_lo[...] + agg_hi[...] + y_ref[...]) + b_ref[...]
    onehot = (bv_ref[...] == lax.broadcasted_iota(jnp.int32, (RB, G), 1))
    onehot = onehot.astype(_f32)
    sums[...] += lax.dot_general(onehot, h, (((0,), (0,)), ((), ())),
                                 preferred_element_type=_f32, precision=_HI)
    cnts[...] += lax.dot_general(onehot, jnp.ones((RB, D), _f32),
                                 (((0,), (0,)), ((), ())),
                                 preferred_element_type=_f32, precision=_HI)

    @pl.when(i == NRB - 1)
    def _():
        pool = sums[...] / jnp.maximum(cnts[...], 1.0)
        out_ref[...] = lax.dot_general(
            pool, wl_ref[...], (((1,), (1,)), ((), ())),
            preferred_element_type=_f32, precision=_HI) + bl_ref[...]


def _final_tc(y, agg, dinv, b3, bv2d, Wl, bl):
    return pl.pallas_call(
        _final_body,
        grid=(NRB,),
        in_specs=[
            pl.BlockSpec((RB, D), lambda i: (i, 0)),
            pl.BlockSpec((RB, D), lambda i: (i, 0)),
            pl.BlockSpec((RB, D), lambda i: (i + NRB, 0)),
            pl.BlockSpec((RB, 1), lambda i: (i, 0)),
            pl.BlockSpec((1, D), lambda i: (0, 0)),
            pl.BlockSpec((RB, 1), lambda i: (i, 0)),
            pl.BlockSpec((C, D), lambda i: (0, 0)),
            pl.BlockSpec((1, C), lambda i: (0, 0)),
        ],
        out_specs=pl.BlockSpec((G, C), lambda i: (0, 0)),
        out_shape=jax.ShapeDtypeStruct((G, C), _f32),
        scratch_shapes=[
            pltpu.VMEM((G, D), _f32),
            pltpu.VMEM((G, D), _f32),
        ],
    )(y, agg, agg, dinv, b3, bv2d, Wl, bl)


def kernel(x, edge_index, batch_vec, W1, b1, W2, b2, W3, b3, Wl, bl):
    src = edge_index[0]
    dst = edge_index[1]
    src3 = src.reshape(NW, NCHUNKS, 1, CHUNK)
    dst3 = dst.reshape(NW, NCHUNKS, 1, CHUNK)
    bv2d = batch_vec.reshape(N, 1)

    deg_lo, deg_hi = _deg_kernel(dst)
    y1, dinv = _prep_tc(deg_lo.reshape(NPAD, 1), deg_hi.reshape(NPAD, 1), x, W1)
    agg1 = _agg_kernel(y1, src3, dst3)
    y2 = _mid_tc(y1, agg1, dinv, b1.reshape(1, D), W2)
    agg2 = _agg_kernel(y2, src3, dst3)
    y3 = _mid_tc(y2, agg2, dinv, b2.reshape(1, D), W3)
    agg3 = _agg_kernel(y3, src3, dst3)
    return _final_tc(y3, agg3, dinv, b3.reshape(1, D), bv2d, Wl,
                     bl.reshape(1, C))


# speedup vs baseline: 1.0290x; 1.0290x over previous
import functools

import jax
import jax.numpy as jnp
from jax import lax
from jax.experimental import pallas as pl
from jax.experimental.pallas import tpu as pltpu
from jax.experimental.pallas import tpu_sc as plsc

N = 10000
E = 320000
D = 128
C = 40
G = 64

NC = 2
NS = 16
NW = NC * NS
EDGES_PER_TILE = E // NW
CHUNK = 80
NCHUNKS = EDGES_PER_TILE // CHUNK
ROW0 = 640
STAGE = 80
NSTAGE0 = ROW0 // STAGE

_MESH = plsc.VectorSubcoreMesh(core_axis_name="c", subcore_axis_name="s")


def _zero_fill(ref, rows, width):
    def body(i, _):
        for j in range(width // 16):
            ref[i, pl.ds(j * 16, 16)] = jnp.zeros((16,), jnp.float32)
        return 0
    lax.fori_loop(0, rows, body, 0, unroll=False)


NPAD = NS * ROW0


def _deg_body(dst_hbm, out_lo, out_hi, idxv, hist, stage_v, sh):
    c = lax.axis_index("c")
    s = lax.axis_index("s")
    w = c * NS + s

    def z(i, _):
        hist[pl.ds(i * 16, 16)] = jnp.zeros((16,), jnp.float32)
        return 0
    lax.fori_loop(0, NPAD // 16, z, 0, unroll=False)

    pltpu.sync_copy(dst_hbm.at[pl.ds(w * EDGES_PER_TILE, EDGES_PER_TILE)], idxv)
    ones16 = jnp.ones((16,), jnp.float32)
    def body(i, _):
        v = idxv[pl.ds(i * 16, 16)]
        plsc.addupdate_scatter(hist, [v], ones16)
        return 0
    lax.fori_loop(0, EDGES_PER_TILE // 16, body, 0, unroll=False)

    pltpu.sync_copy(hist, sh.at[pl.ds(s * NPAD, NPAD)])
    plsc.subcore_barrier()
    for t in range(NS):
        pltpu.sync_copy(sh.at[pl.ds(t * NPAD + s * ROW0, ROW0)],
                        hist.at[pl.ds(t * ROW0, ROW0)])
    def red(j, _):
        acc = hist[pl.ds(j * 16, 16)]
        for t in range(1, NS):
            acc = acc + hist[pl.ds(t * ROW0 + j * 16, 16)]
        stage_v[pl.ds(j * 16, 16)] = acc
        return 0
    lax.fori_loop(0, ROW0 // 16, red, 0, unroll=False)

    @pl.when(c == 0)
    def _():
        pltpu.sync_copy(stage_v, out_lo.at[pl.ds(s * ROW0, ROW0)])

    @pl.when(c == 1)
    def _():
        pltpu.sync_copy(stage_v, out_hi.at[pl.ds(s * ROW0, ROW0)])


_deg_kernel = functools.partial(
    pl.kernel,
    mesh=_MESH,
    compiler_params=pltpu.CompilerParams(needs_layout_passes=False),
    out_type=[jax.ShapeDtypeStruct((NPAD,), jnp.float32),
              jax.ShapeDtypeStruct((NPAD,), jnp.float32)],
    scratch_types=[
        pltpu.VMEM((EDGES_PER_TILE,), jnp.int32),
        pltpu.VMEM((NPAD,), jnp.float32),
        pltpu.VMEM((ROW0,), jnp.float32),
        pltpu.VMEM_SHARED((NS * NPAD,), jnp.float32),
    ],
)(_deg_body)


def _agg_body(y_hbm, src_hbm, dst3_hbm, out_hbm, srcall, dstv0, dstv1, dstv2,
              rows0, rows1, rows2, agg_sh,
              gsem0, gsem1, gsem2, ssem0, ssem1, ssem2, dsem0, dsem1, dsem2):
    c = lax.axis_index("c")
    s = lax.axis_index("s")
    w = c * NS + s
    nstage = jnp.where(s == NS - 1, 5, NSTAGE0)

    _zero_fill(rows0, STAGE, D)
    def zc(k, _):
        pltpu.sync_copy(rows0, agg_sh.at[pl.ds(s * ROW0 + k * STAGE, STAGE)])
        return 0
    lax.fori_loop(0, nstage, zc, 0, unroll=False)

    pltpu.sync_copy(src_hbm.at[pl.ds(w * EDGES_PER_TILE, EDGES_PER_TILE)], srcall)
    plsc.subcore_barrier()

    rows = (rows0, rows1, rows2)
    dstv = (dstv0, dstv1, dstv2)
    gs = (gsem0, gsem1, gsem2)
    ss = (ssem0, ssem1, ssem2)
    dsm = (dsem0, dsem1, dsem2)

    def g_start(k, b):
        pltpu.async_copy(y_hbm.at[srcall.at[pl.ds(k * CHUNK, CHUNK)]], rows[b], gs[b])
    def g_wait(b):
        pltpu.make_async_copy(y_hbm.at[srcall.at[pl.ds(0, CHUNK)]], rows[b], gs[b]).wait()
    def d_start(k, b):
        pltpu.async_copy(dst3_hbm.at[w, k], dstv[b], dsm[b])
    def d_wait(b):
        pltpu.make_async_copy(dst3_hbm.at[w, 0], dstv[b], dsm[b]).wait()
    def s_start(b):
        pltpu.async_copy(rows[b], agg_sh.at[dstv[b]], ss[b], add=True)
    def s_wait(b):
        pltpu.make_async_copy(rows[b], agg_sh.at[dstv[b]], ss[b]).wait()

    d_start(0, 0); g_start(0, 0); d_start(1, 1); g_start(1, 1)
    g_wait(0); d_wait(0); s_start(0); d_start(2, 2); g_start(2, 2)
    g_wait(1); d_wait(1); s_start(1); s_wait(0); d_start(3, 0); g_start(3, 0)
    g_wait(2); d_wait(2); s_start(2); s_wait(1); d_start(4, 1); g_start(4, 1)

    def triple(p, _):
        for b in (0, 1, 2):
            k = 3 * p + b
            bn = (b + 2) % 3
            g_wait(b); d_wait(b); s_start(b)
            s_wait(bn); d_start(k + 2, bn); g_start(k + 2, bn)
        return 0
    lax.fori_loop(1, (NCHUNKS - 2) // 3, triple, 0, unroll=False)

    g_wait(0); d_wait(0); s_start(0)
    g_wait(1); d_wait(1); s_start(1)
    s_wait(2); s_wait(0); s_wait(1)
    plsc.subcore_barrier()

    outb = (rows0, rows1)
    def ld_start(k, b):
        pltpu.async_copy(agg_sh.at[pl.ds(s * ROW0 + k * STAGE, STAGE)],
                         outb[b], gs[b])
    def ld_wait(b):
        pltpu.make_async_copy(agg_sh.at[pl.ds(s * ROW0, STAGE)],
                              outb[b], gs[b]).wait()
    def st_start(k, b):
        pltpu.async_copy(outb[b], out_hbm.at[pl.ds(c * N + s * ROW0 + k * STAGE,
                                                   STAGE)], ss[b])
    def st_wait(b):
        pltpu.make_async_copy(outb[b], out_hbm.at[pl.ds(c * N + s * ROW0,
                                                        STAGE)], ss[b]).wait()

    ld_start(0, 0)
    for k in range(NSTAGE0):
        b = k % 2
        @pl.when(k < nstage)
        def _():
            ld_wait(b)
            @pl.when(k + 1 < nstage)
            def _():
                if k >= 1:
                    st_wait(1 - b)
                ld_start(k + 1, 1 - b)
            st_start(k, b)
    st_wait(0); st_wait(1)


_agg_kernel = functools.partial(
    pl.kernel,
    mesh=_MESH,
    out_type=jax.ShapeDtypeStruct((NC * N, D), jnp.float32),
    scratch_types=[
        pltpu.VMEM((EDGES_PER_TILE,), jnp.int32),
        pltpu.VMEM((CHUNK,), jnp.int32),
        pltpu.VMEM((CHUNK,), jnp.int32),
        pltpu.VMEM((CHUNK,), jnp.int32),
        pltpu.VMEM((CHUNK, D), jnp.float32),
        pltpu.VMEM((CHUNK, D), jnp.float32),
        pltpu.VMEM((CHUNK, D), jnp.float32),
        pltpu.VMEM_SHARED((N, D), jnp.float32),
        pltpu.SemaphoreType.DMA, pltpu.SemaphoreType.DMA, pltpu.SemaphoreType.DMA,
        pltpu.SemaphoreType.DMA, pltpu.SemaphoreType.DMA, pltpu.SemaphoreType.DMA,
        pltpu.SemaphoreType.DMA, pltpu.SemaphoreType.DMA, pltpu.SemaphoreType.DMA,
    ],
)(_agg_body)


RB = 1000
NRB = N // RB

_f32 = jnp.float32
_HI = lax.Precision.HIGHEST


def _prep_body(deg_lo, deg_hi, x_ref, w_ref, y_out, dinv_out):
    deg = deg_lo[...] + deg_hi[...] + 1.0
    dinv = lax.rsqrt(deg)
    xw = lax.dot_general(x_ref[...], w_ref[...], (((1,), (1,)), ((), ())),
                         preferred_element_type=_f32, precision=_HI)
    y_out[...] = xw * dinv
    dinv_out[...] = dinv


def _prep_tc(deg_lo, deg_hi, x, W1):
    return pl.pallas_call(
        _prep_body,
        grid=(NRB,),
        in_specs=[
            pl.BlockSpec((RB, 1), lambda i: (i, 0)),
            pl.BlockSpec((RB, 1), lambda i: (i, 0)),
            pl.BlockSpec((RB, D), lambda i: (i, 0)),
            pl.BlockSpec((D, D), lambda i: (0, 0)),
        ],
        out_specs=[
            pl.BlockSpec((RB, D), lambda i: (i, 0)),
            pl.BlockSpec((RB, 1), lambda i: (i, 0)),
        ],
        out_shape=[
            jax.ShapeDtypeStruct((N, D), _f32),
            jax.ShapeDtypeStruct((N, 1), _f32),
        ],
    )(deg_lo, deg_hi, x, W1)


def _mid_body(y_ref, agg_lo, agg_hi, dinv_ref, b_ref, w_ref, out_ref):
    dinv = dinv_ref[...]
    h = dinv * (agg_lo[...] + agg_hi[...] + y_ref[...]) + b_ref[...]
    h = jnp.maximum(h, 0.0)
    xw = lax.dot_general(h, w_ref[...], (((1,), (1,)), ((), ())),
                         preferred_element_type=_f32, precision=_HI)
    out_ref[...] = xw * dinv


def _mid_tc(y, agg, dinv, b, Wn):
    return pl.pallas_call(
        _mid_body,
        grid=(NRB,),
        in_specs=[
            pl.BlockSpec((RB, D), lambda i: (i, 0)),
            pl.BlockSpec((RB, D), lambda i: (i, 0)),
            pl.BlockSpec((RB, D), lambda i: (i + NRB, 0)),
            pl.BlockSpec((RB, 1), lambda i: (i, 0)),
            pl.BlockSpec((1, D), lambda i: (0, 0)),
            pl.BlockSpec((D, D), lambda i: (0, 0)),
        ],
        out_specs=pl.BlockSpec((RB, D), lambda i: (i, 0)),
        out_shape=jax.ShapeDtypeStruct((N, D), _f32),
    )(y, agg, agg, dinv, b, Wn)


def _final_body(y_ref, agg_lo, agg_hi, dinv_ref, b_ref, bv_ref, wl_ref,
                bl_ref, out_ref, sums, cnts):
    i = pl.program_id(0)

    @pl.when(i == 0)
    def _():
        sums[...] = jnp.zeros_like(sums)
        cnts[...] = jnp.zeros_like(cnts)

    h = dinv_ref[...] * (agg_lo[...] + agg_hi[...] + y_ref[...]) + b_ref[...]
    onehot = (bv_ref[...] == lax.broadcasted_iota(jnp.int32, (RB, G), 1))
    onehot = onehot.astype(_f32)
    sums[...] += lax.dot_general(onehot, h, (((0,), (0,)), ((), ())),
                                 preferred_element_type=_f32, precision=_HI)
    cnts[...] += lax.dot_general(onehot, jnp.ones((RB, D), _f32),
                                 (((0,), (0,)), ((), ())),
                                 preferred_element_type=_f32, precision=_HI)

    @pl.when(i == NRB - 1)
    def _():
        pool = sums[...] / jnp.maximum(cnts[...], 1.0)
        out_ref[...] = lax.dot_general(
            pool, wl_ref[...], (((1,), (1,)), ((), ())),
            preferred_element_type=_f32, precision=_HI) + bl_ref[...]


def _final_tc(y, agg, dinv, b3, bv2d, Wl, bl):
    return pl.pallas_call(
        _final_body,
        grid=(NRB,),
        in_specs=[
            pl.BlockSpec((RB, D), lambda i: (i, 0)),
            pl.BlockSpec((RB, D), lambda i: (i, 0)),
            pl.BlockSpec((RB, D), lambda i: (i + NRB, 0)),
            pl.BlockSpec((RB, 1), lambda i: (i, 0)),
            pl.BlockSpec((1, D), lambda i: (0, 0)),
            pl.BlockSpec((RB, 1), lambda i: (i, 0)),
            pl.BlockSpec((C, D), lambda i: (0, 0)),
            pl.BlockSpec((1, C), lambda i: (0, 0)),
        ],
        out_specs=pl.BlockSpec((G, C), lambda i: (0, 0)),
        out_shape=jax.ShapeDtypeStruct((G, C), _f32),
        scratch_shapes=[
            pltpu.VMEM((G, D), _f32),
            pltpu.VMEM((G, D), _f32),
        ],
    )(y, agg, agg, dinv, b3, bv2d, Wl, bl)


def kernel(x, edge_index, batch_vec, W1, b1, W2, b2, W3, b3, Wl, bl):
    src = edge_index[0]
    dst = edge_index[1]
    dst3 = dst.reshape(NW, NCHUNKS, CHUNK)
    bv2d = batch_vec.reshape(N, 1)

    deg_lo, deg_hi = _deg_kernel(dst)
    y1, dinv = _prep_tc(deg_lo.reshape(NPAD, 1), deg_hi.reshape(NPAD, 1), x, W1)
    agg1 = _agg_kernel(y1, src, dst3)
    y2 = _mid_tc(y1, agg1, dinv, b1.reshape(1, D), W2)
    agg2 = _agg_kernel(y2, src, dst3)
    y3 = _mid_tc(y2, agg2, dinv, b2.reshape(1, D), W3)
    agg3 = _agg_kernel(y3, src, dst3)
    return _final_tc(y3, agg3, dinv, b3.reshape(1, D), bv2d, Wl,
                     bl.reshape(1, C))
